# baseline (speedup 1.0000x reference)
import math

import jax
import jax.numpy as jnp
import numpy as np
from jax import lax
from jax.experimental import pallas as pl
from jax.experimental.pallas import tpu as pltpu
from jax.experimental.pallas import tpu_sc as plsc

D_MODEL = 128
MAX_LEN = 1500
VOCAB = 28
BATCH = 256
SEQ = 1024

NC, NS, LANES = 2, 16, 16
NW = NC * NS
B_SC = 128
TOK_PER_W = B_SC * SEQ // NW
QUARTERS = 4
Q = SEQ // QUARTERS
QD = Q * D_MODEL
SEQ_PER_W = TOK_PER_W // SEQ
BB = 8


def _make_pe_np(max_len, d_model):
    position = np.arange(0, max_len, dtype=np.float32)[:, None]
    div_term = np.exp(
        np.arange(0, d_model, 2).astype(np.float32) * -(math.log(10000.0) / d_model)
    )
    pe = np.zeros((max_len, d_model), dtype=np.float32)
    pe[:, 0::2] = np.sin(position * div_term)
    pe[:, 1::2] = np.cos(position * div_term)
    return pe


def _interleave32_np(x2d):
    n = x2d.shape[0]
    g = x2d.reshape(n, D_MODEL // 32, 2, 16)
    m = np.empty((n, D_MODEL // 32, 16, 2), dtype=x2d.dtype)
    m[..., 0] = g[:, :, 0, :]
    m[..., 1] = g[:, :, 1, :]
    return m.reshape(n * D_MODEL)


_PE_NP = _interleave32_np(_make_pe_np(MAX_LEN, D_MODEL)[:SEQ])


def _sc_embed(tokens_flat, table_bf, pe_bf):
    mesh = plsc.VectorSubcoreMesh(
        core_axis_name="c", subcore_axis_name="s", num_cores=NC, num_subcores=NS
    )

    def body(tok_hbm, table_hbm, pe_hbm, out_hbm,
             table_v, pe0, pe1, idx_v, rows0, rows1,
             sem0, sem1, psem0, psem1):
        wid = lax.axis_index("s") * NC + lax.axis_index("c")
        base = wid * TOK_PER_W
        pes = (pe0, pe1)
        psems = (psem0, psem1)
        pltpu.async_copy(pe_hbm.at[pl.ds(0, QD // 2)], pe0, psem0)
        pltpu.sync_copy(table_hbm, table_v)
        pltpu.sync_copy(tok_hbm.at[pl.ds(base, TOK_PER_W)], idx_v)
        rows = (rows0, rows1)
        sems = (sem0, sem1)

        def compute_chunk(loc, rows_b, pe_v):
            @plsc.parallel_loop(0, Q // LANES, unroll=1)
            def _rb_body(rb):
                tokv = idx_v[pl.ds(loc + rb * LANES, LANES)] * (D_MODEL // 2)
                gbase = rb * (LANES * D_MODEL)
                nj = D_MODEL // 32
                for lane in range(LANES):
                    tbase = tokv[lane]
                    rbase = gbase + lane * D_MODEL
                    pbase = (gbase // 2) + lane * (D_MODEL // 2)
                    tvi = [table_v[pl.ds(tbase + j * LANES, LANES)] for j in range(nj)]
                    pvi = [pe_v[pl.ds(pbase + j * LANES, LANES)] for j in range(nj)]
                    sums = [
                        plsc.bitcast(tvi[j], jnp.bfloat16)
                        + plsc.bitcast(pvi[j], jnp.bfloat16)
                        for j in range(nj)
                    ]
                    for j in range(nj):
                        s0, s1 = plsc.unpack(
                            sums[j],
                            format=plsc.PackFormat.INTERLEAVED,
                            preferred_element_type=jnp.float32,
                        )
                        off = rbase + j * 32
                        rows_b[pl.ds(off, LANES)] = s0
                        rows_b[pl.ds(off + LANES, LANES)] = s1

        for q in range(QUARTERS):
            pe_v = pes[q % 2]
            pltpu.make_async_copy(
                pe_hbm.at[pl.ds(0, QD // 2)], pe_v, psems[q % 2]
            ).wait()
            if q + 1 < QUARTERS:
                pltpu.async_copy(
                    pe_hbm.at[pl.ds((q + 1) * (QD // 2), QD // 2)],
                    pes[(q + 1) % 2],
                    psems[(q + 1) % 2],
                )

            def s2_body(s2, _, q=q, pe_v=pe_v):
                for b in range(2):
                    s = s2 * 2 + b
                    g = base + s * SEQ + q * Q

                    if q == 0:
                        @pl.when(s2 > 0)
                        def _wait(b=b):
                            pltpu.make_async_copy(
                                rows[b], out_hbm.at[pl.ds(0, QD)], sems[b]
                            ).wait()
                    else:
                        pltpu.make_async_copy(
                            rows[b], out_hbm.at[pl.ds(0, QD)], sems[b]
                        ).wait()

                    compute_chunk(s * SEQ + q * Q, rows[b], pe_v)
                    pltpu.async_copy(
                        rows[b], out_hbm.at[pl.ds(g * D_MODEL, QD)], sems[b]
                    )
                return 0

            lax.fori_loop(0, SEQ_PER_W // 2, s2_body, 0)
        for b in range(2):
            pltpu.make_async_copy(
                rows[b], out_hbm.at[pl.ds(0, QD)], sems[b]
            ).wait()

    run = pl.kernel(
        body,
        out_type=jax.ShapeDtypeStruct((BATCH * SEQ * D_MODEL,), jnp.float32),
        mesh=mesh,
        compiler_params=pltpu.CompilerParams(needs_layout_passes=False),
        scratch_types=[
            pltpu.VMEM((VOCAB * D_MODEL // 2,), jnp.int32),
            pltpu.VMEM((QD // 2,), jnp.int32),
            pltpu.VMEM((QD // 2,), jnp.int32),
            pltpu.VMEM((TOK_PER_W,), jnp.int32),
            pltpu.VMEM((QD,), jnp.float32),
            pltpu.VMEM((QD,), jnp.float32),
            pltpu.SemaphoreType.DMA,
            pltpu.SemaphoreType.DMA,
            pltpu.SemaphoreType.DMA,
            pltpu.SemaphoreType.DMA,
        ],
    )
    return run(tokens_flat, table_bf, pe_bf)


def _tc_half(tokens, table, pe2d, sc_out):
    nblk = (BATCH - B_SC) // BB
    blk0 = B_SC // BB

    def tck(tok_ref, table_ref, pe_ref, _sc_ref, out_ref):
        pe = pe_ref[...]
        tbl = table_ref[...]
        for s in range(BB):
            tok1 = tok_ref[s, :]
            oh = (
                lax.broadcast_in_dim(tok1, (SEQ, VOCAB), (0,))
                == lax.broadcasted_iota(jnp.int32, (SEQ, VOCAB), 1)
            ).astype(jnp.float32)
            out_ref[s, :, :] = (
                jnp.dot(oh, tbl, preferred_element_type=jnp.float32) + pe
            )

    return pl.pallas_call(
        tck,
        grid=(nblk,),
        in_specs=[
            pl.BlockSpec((BB, SEQ), lambda i: (blk0 + i, 0)),
            pl.BlockSpec((VOCAB, D_MODEL), lambda i: (0, 0)),
            pl.BlockSpec((SEQ, D_MODEL), lambda i: (0, 0)),
            pl.BlockSpec(memory_space=pl.ANY),
        ],
        out_specs=pl.BlockSpec((BB, SEQ, D_MODEL), lambda i: (blk0 + i, 0, 0)),
        out_shape=jax.ShapeDtypeStruct((BATCH, SEQ, D_MODEL), jnp.float32),
        input_output_aliases={3: 0},
    )(tokens, table, pe2d, sc_out)


def kernel(tokens, table):
    tokens = tokens.astype(jnp.int32)
    tokens_flat = tokens.reshape(-1)
    table_il = (
        table.reshape(VOCAB, D_MODEL // 32, 2, 16)
        .transpose(0, 1, 3, 2)
        .reshape(-1)
    )
    table_bf = table_il.astype(jnp.bfloat16)
    pe_bf = jnp.asarray(_PE_NP).astype(jnp.bfloat16)
    table_i = lax.bitcast_convert_type(table_bf.reshape(-1, 2), jnp.int32)
    pe_i = lax.bitcast_convert_type(pe_bf.reshape(-1, 2), jnp.int32)
    sc_out = _sc_embed(tokens_flat, table_i, pe_i).reshape(BATCH, SEQ, D_MODEL)
    pe2d = jnp.asarray(
        _make_pe_np(MAX_LEN, D_MODEL)[:SEQ]
    )
    return _tc_half(tokens, table, pe2d, sc_out)


# speedup vs baseline: 1.0403x; 1.0403x over previous
import math

import jax
import jax.numpy as jnp
import numpy as np
from jax import lax
from jax.experimental import pallas as pl
from jax.experimental.pallas import tpu as pltpu
from jax.experimental.pallas import tpu_sc as plsc

D_MODEL = 128
MAX_LEN = 1500
VOCAB = 28
BATCH = 256
SEQ = 1024

NC, NS, LANES = 2, 16, 16
NW = NC * NS
TOK_PER_W = BATCH * SEQ // NW
QUARTERS = 4
Q = SEQ // QUARTERS
QD = Q * D_MODEL
SEQ_PER_W = TOK_PER_W // SEQ


def _make_pe_np(max_len, d_model):
    position = np.arange(0, max_len, dtype=np.float32)[:, None]
    div_term = np.exp(
        np.arange(0, d_model, 2).astype(np.float32) * -(math.log(10000.0) / d_model)
    )
    pe = np.zeros((max_len, d_model), dtype=np.float32)
    pe[:, 0::2] = np.sin(position * div_term)
    pe[:, 1::2] = np.cos(position * div_term)
    return pe


def _interleave32_np(x2d):
    n = x2d.shape[0]
    g = x2d.reshape(n, D_MODEL // 32, 2, 16)
    m = np.empty((n, D_MODEL // 32, 16, 2), dtype=x2d.dtype)
    m[..., 0] = g[:, :, 0, :]
    m[..., 1] = g[:, :, 1, :]
    return m.reshape(n * D_MODEL)


_PE_NP = _interleave32_np(_make_pe_np(MAX_LEN, D_MODEL)[:SEQ])


def _sc_embed(tokens_flat, table_bf, pe_bf):
    mesh = plsc.VectorSubcoreMesh(
        core_axis_name="c", subcore_axis_name="s", num_cores=NC, num_subcores=NS
    )

    def body(tok_hbm, table_hbm, pe_hbm, out_hbm,
             table_v, pe0, pe1, idx_v, rows0, rows1,
             sem0, sem1, psem0, psem1):
        wid = lax.axis_index("s") * NC + lax.axis_index("c")
        base = wid * TOK_PER_W
        pes = (pe0, pe1)
        psems = (psem0, psem1)
        pltpu.async_copy(pe_hbm.at[pl.ds(0, QD // 2)], pe0, psem0)
        pltpu.sync_copy(table_hbm, table_v)
        pltpu.sync_copy(tok_hbm.at[pl.ds(base, TOK_PER_W)], idx_v)
        rows = (rows0, rows1)
        sems = (sem0, sem1)

        def compute_chunk(loc, rows_b, pe_v):
            @plsc.parallel_loop(0, Q // LANES, unroll=1)
            def _rb_body(rb):
                tokv = idx_v[pl.ds(loc + rb * LANES, LANES)] * (D_MODEL // 2)
                gbase = rb * (LANES * D_MODEL)
                nj = D_MODEL // 32
                for lane in range(LANES):
                    tbase = tokv[lane]
                    rbase = gbase + lane * D_MODEL
                    pbase = (gbase // 2) + lane * (D_MODEL // 2)
                    tvi = [table_v[pl.ds(tbase + j * LANES, LANES)] for j in range(nj)]
                    pvi = [pe_v[pl.ds(pbase + j * LANES, LANES)] for j in range(nj)]
                    sums = [
                        plsc.bitcast(tvi[j], jnp.bfloat16)
                        + plsc.bitcast(pvi[j], jnp.bfloat16)
                        for j in range(nj)
                    ]
                    for j in range(nj):
                        s0, s1 = plsc.unpack(
                            sums[j],
                            format=plsc.PackFormat.INTERLEAVED,
                            preferred_element_type=jnp.float32,
                        )
                        off = rbase + j * 32
                        rows_b[pl.ds(off, LANES)] = s0
                        rows_b[pl.ds(off + LANES, LANES)] = s1

        for q in range(QUARTERS):
            pe_v = pes[q % 2]
            pltpu.make_async_copy(
                pe_hbm.at[pl.ds(0, QD // 2)], pe_v, psems[q % 2]
            ).wait()
            if q + 1 < QUARTERS:
                pltpu.async_copy(
                    pe_hbm.at[pl.ds((q + 1) * (QD // 2), QD // 2)],
                    pes[(q + 1) % 2],
                    psems[(q + 1) % 2],
                )

            def s2_body(s2, _, q=q, pe_v=pe_v):
                for b in range(2):
                    s = s2 * 2 + b
                    g = base + s * SEQ + q * Q

                    if q == 0:
                        @pl.when(s2 > 0)
                        def _wait(b=b):
                            pltpu.make_async_copy(
                                rows[b], out_hbm.at[pl.ds(0, QD)], sems[b]
                            ).wait()
                    else:
                        pltpu.make_async_copy(
                            rows[b], out_hbm.at[pl.ds(0, QD)], sems[b]
                        ).wait()

                    compute_chunk(s * SEQ + q * Q, rows[b], pe_v)
                    pltpu.async_copy(
                        rows[b], out_hbm.at[pl.ds(g * D_MODEL, QD)], sems[b]
                    )
                return 0

            lax.fori_loop(0, SEQ_PER_W // 2, s2_body, 0)
        for b in range(2):
            pltpu.make_async_copy(
                rows[b], out_hbm.at[pl.ds(0, QD)], sems[b]
            ).wait()

    run = pl.kernel(
        body,
        out_type=jax.ShapeDtypeStruct((BATCH * SEQ * D_MODEL,), jnp.float32),
        mesh=mesh,
        compiler_params=pltpu.CompilerParams(needs_layout_passes=False),
        scratch_types=[
            pltpu.VMEM((VOCAB * D_MODEL // 2,), jnp.int32),
            pltpu.VMEM((QD // 2,), jnp.int32),
            pltpu.VMEM((QD // 2,), jnp.int32),
            pltpu.VMEM((TOK_PER_W,), jnp.int32),
            pltpu.VMEM((QD,), jnp.float32),
            pltpu.VMEM((QD,), jnp.float32),
            pltpu.SemaphoreType.DMA,
            pltpu.SemaphoreType.DMA,
            pltpu.SemaphoreType.DMA,
            pltpu.SemaphoreType.DMA,
        ],
    )
    return run(tokens_flat, table_bf, pe_bf)


def kernel(tokens, table):
    tokens_flat = tokens.reshape(-1).astype(jnp.int32)
    table_il = (
        table.reshape(VOCAB, D_MODEL // 32, 2, 16)
        .transpose(0, 1, 3, 2)
        .reshape(-1)
    )
    table_bf = table_il.astype(jnp.bfloat16)
    pe_bf = jnp.asarray(_PE_NP).astype(jnp.bfloat16)
    table_i = lax.bitcast_convert_type(table_bf.reshape(-1, 2), jnp.int32)
    pe_i = lax.bitcast_convert_type(pe_bf.reshape(-1, 2), jnp.int32)
    out = _sc_embed(tokens_flat, table_i, pe_i)
    return out.reshape(BATCH, SEQ, D_MODEL)
